# baseline (speedup 1.0000x reference)
import functools

import jax
import jax.numpy as jnp
from jax import lax
from jax.experimental import pallas as pl
from jax.experimental.pallas import tpu as pltpu
from jax.experimental.pallas import tpu_sc as plsc

N = 50000
K = 50000
S = 800000

NC, NS, LANES = 2, 16, 16
NW = NC * NS
RPT = 1568
NPAD = NW * RPT

CE = 1280
NCH_E = S // CE
ECH_PER_TILE = -(-NCH_E // NW)
SUBB = 128

CH = 1600
NCH_M = S // CH
G = 128

_SC_PARAMS = dict(
    compiler_params=pltpu.CompilerParams(
        use_tc_tiling_on_sc=False, needs_layout_passes=False
    ),
)


def _mesh():
    return plsc.VectorSubcoreMesh(
        core_axis_name="c", subcore_axis_name="s", num_cores=NC, num_subcores=NS
    )


def _wid():
    return lax.axis_index("s") * NC + lax.axis_index("c")


def _tc_pr_body(f_ref, c_ref, w1_ref, b1_ref, p_ref, rn_ref):
    c = c_ref[...]
    w1 = w1_ref[...]
    r = (c[:, 0:1] * w1[1:2, :] + c[:, 1:2] * w1[2:3, :] + c[:, 2:3] * w1[3:4, :])
    rn_ref[...] = -r
    p_ref[...] = r + f_ref[...] * w1[0:1, :] + b1_ref[...]


def _tc_pr(f_pad, c_pad, W1, b1):
    blk = 1024
    return pl.pallas_call(
        _tc_pr_body,
        grid=(NPAD // blk,),
        in_specs=[
            pl.BlockSpec((blk, 1), lambda i: (i, 0)),
            pl.BlockSpec((blk, 3), lambda i: (i, 0)),
            pl.BlockSpec((4, 32), lambda i: (0, 0)),
            pl.BlockSpec((1, 32), lambda i: (0, 0)),
        ],
        out_specs=[pl.BlockSpec((blk, 32), lambda i: (i, 0))] * 2,
        out_shape=[jax.ShapeDtypeStruct((NPAD, 32), jnp.float32)] * 2,
    )(f_pad, c_pad, W1, b1.reshape(1, 32))


def _sc_q_body(rn_hbm, kp_hbm, qn_hbm, idx_v, rows_v, sem):
    base = _wid() * RPT
    pltpu.sync_copy(kp_hbm.at[pl.ds(base, RPT)], idx_v)
    descs = []
    for b in range(RPT // 112):
        descs.append(
            pltpu.async_copy(
                rn_hbm.at[idx_v.at[pl.ds(b * 112, 112)]],
                rows_v.at[pl.ds(b * 112, 112)],
                sem,
            )
        )
    for d in descs:
        d.wait()
    pltpu.sync_copy(rows_v, qn_hbm.at[pl.ds(base, RPT)])


def _sc_q(RN, kp_pad):
    return pl.kernel(
        _sc_q_body,
        out_type=jax.ShapeDtypeStruct((NPAD, 32), jnp.float32),
        mesh=_mesh(),
        scratch_types=[
            pltpu.VMEM((RPT,), jnp.int32),
            pltpu.VMEM((RPT, 32), jnp.float32),
            pltpu.SemaphoreType.DMA,
        ],
        **_SC_PARAMS,
    )(RN, kp_pad)


def _sc_edge_body(p_hbm, qn_hbm, src_hbm, dst_hbm, e_hbm,
                  sidx, didx, eb, sem_i, sem_p, sem_q):
    w = _wid()

    def chunk_body(k, carry):
        c = w + k * NW

        @pl.when(c < NCH_E)
        def _():
            ebase = c * CE
            di = pltpu.async_copy(src_hbm.at[pl.ds(ebase, CE)], sidx, sem_i)
            dj = pltpu.async_copy(dst_hbm.at[pl.ds(ebase, CE)], didx, sem_i)
            di.wait()
            dj.wait()
            descs = []
            for b in range(CE // SUBB):
                descs.append(pltpu.async_copy(
                    p_hbm.at[sidx.at[pl.ds(b * SUBB, SUBB)]],
                    eb.at[pl.ds(b * SUBB, SUBB)], sem_p))
            for d in descs:
                d.wait()
            descs = []
            for b in range(CE // SUBB):
                d = pltpu.make_async_copy(
                    qn_hbm.at[didx.at[pl.ds(b * SUBB, SUBB)]],
                    eb.at[pl.ds(b * SUBB, SUBB)], sem_q)
                d.start(add=True)
                descs.append(d)
            for d in descs:
                d.wait()
            pltpu.sync_copy(eb, e_hbm.at[pl.ds(ebase, CE)])

        return carry

    lax.fori_loop(0, ECH_PER_TILE, chunk_body, 0)


def _sc_edge(P, QN, src, dst):
    return pl.kernel(
        _sc_edge_body,
        out_type=jax.ShapeDtypeStruct((S, 32), jnp.float32),
        mesh=_mesh(),
        scratch_types=[
            pltpu.VMEM((CE,), jnp.int32),
            pltpu.VMEM((CE,), jnp.int32),
            pltpu.VMEM((CE, 32), jnp.float32),
            pltpu.SemaphoreType.DMA,
            pltpu.SemaphoreType.DMA,
            pltpu.SemaphoreType.DMA,
        ],
        **_SC_PARAMS,
    )(P, QN, src, dst)


def _tc_h_body(e_ref, w2_ref, b2_ref, h_ref):
    x = jnp.maximum(e_ref[...], 0.0)
    h = jnp.dot(x, w2_ref[...], preferred_element_type=jnp.float32)
    h = jnp.maximum(h + b2_ref[...], 0.0)
    h_ref[...] = h.reshape(h_ref.shape)


def _tc_h(E4, W2bd, b2t):
    blk = 2000
    return pl.pallas_call(
        _tc_h_body,
        grid=(S // 4 // blk,),
        in_specs=[
            pl.BlockSpec((blk, 128), lambda i: (i, 0)),
            pl.BlockSpec((128, 256), lambda i: (0, 0)),
            pl.BlockSpec((1, 256), lambda i: (0, 0)),
        ],
        out_specs=pl.BlockSpec((2 * blk, 128), lambda i: (i, 0)),
        out_shape=jax.ShapeDtypeStruct((S // 2, 128), jnp.float32),
    )(E4, W2bd, b2t.reshape(1, 256))


def _sc_smax_body(h_hbm, dst_hbm, agg_hbm, table, dbuf0, dbuf1, ids0, ids1,
                  dloc0, dloc1, hbuf0, hbuf1, sem_d, sem_h, sem_h2):
    w = _wid()
    lo = w * RPT
    zf = jnp.zeros((16,), jnp.float32)
    zi = jnp.zeros((16,), jnp.int32)

    def zrow(r, carry):
        for j in range(4):
            table[r, pl.ds(j * 16, 16)] = zf
        return carry

    lax.fori_loop(0, RPT, zrow, 0)

    def zids(v, carry):
        ids0[pl.ds(v * 16, 16)] = zi
        ids1[pl.ds(v * 16, 16)] = zi
        return carry

    lax.fori_loop(0, CH // 16, zids, 0)

    iota = lax.iota(jnp.int32, 16)
    urpt = jnp.uint32(RPT)

    pltpu.async_copy(dst_hbm.at[pl.ds(0, CH)], dbuf0, sem_d)

    def filter_chunk(cidx, dbuf_cur, dbuf_nxt, ids_c, dloc_c):
        ebase = cidx * CH
        pltpu.make_async_copy(dst_hbm.at[pl.ds(ebase, CH)], dbuf_cur, sem_d).wait()

        @pl.when(cidx + 1 < NCH_M)
        def _():
            pltpu.async_copy(
                dst_hbm.at[pl.ds((cidx + 1) * CH, CH)], dbuf_nxt, sem_d)

        base_eid = ebase + iota

        def filt(v, m_v):
            for u in range(2):
                vv = v * 2 + u
                d16 = dbuf_cur[pl.ds(vv * 16, 16)]
                dl = d16 - lo
                msk = dl.astype(jnp.uint32) < urpt
                eid = base_eid + vv * 16
                csum = plsc.cumsum(msk.astype(jnp.int32))
                pos = csum + (m_v - 1)
                plsc.store_scatter(ids_c, [pos], eid, mask=msk)
                plsc.store_scatter(dloc_c, [pos], dl, mask=msk)
                m_v = m_v + plsc.all_reduce_population_count(msk)
            return m_v

        m_v = lax.fori_loop(0, CH // 32, filt, jnp.zeros((16,), jnp.int32))
        return jnp.max(m_v)

    def upd_span(hbuf, dloc_c, off, cnt):
        def body1(i, carry3):
            d = jnp.max(plsc.load_gather(
                dloc_c, [jnp.full((16,), off + i, jnp.int32)]))
            for j in range(4):
                t = table[d, pl.ds(j * 16, 16)]
                hh = hbuf[i, pl.ds(j * 16, 16)]
                table[d, pl.ds(j * 16, 16)] = jnp.maximum(t, hh)
            return carry3

        def bodyq(q, carry3):
            for k in range(4):
                body1(q * 4 + k, 0)
            return carry3

        nq = cnt >> 2
        lax.fori_loop(0, nq, bodyq, 0)
        lax.fori_loop(nq * 4, cnt, body1, 0)

    def update_chunk(m, ids_c, dloc_c, hbuf_c):
        upd_span(hbuf_c, dloc_c, 0, jnp.minimum(G, m))
        nb = (m + (G - 1)) >> 7

        def sub(b, carry2):
            off = b * G
            cnt = jnp.minimum(G, m - off)
            pltpu.async_copy(
                h_hbm.at[ids_c.at[pl.ds(off, G)]], hbuf_c, sem_h2).wait()
            upd_span(hbuf_c, dloc_c, off, cnt)
            return carry2

        lax.fori_loop(1, nb, sub, 0)

    def half_step(cidx, m_prev, dbuf_cur, dbuf_nxt, ids_c, dloc_c, hbuf_c,
                  ids_p, dloc_p, hbuf_p):
        m = filter_chunk(cidx, dbuf_cur, dbuf_nxt, ids_c, dloc_c)

        @pl.when(m_prev > 0)
        def _():
            pltpu.make_async_copy(
                h_hbm.at[ids_p.at[pl.ds(0, G)]], hbuf_p, sem_h).wait()

        @pl.when(m > 0)
        def _():
            pltpu.async_copy(h_hbm.at[ids_c.at[pl.ds(0, G)]], hbuf_c, sem_h)

        @pl.when(m_prev > 0)
        def _():
            update_chunk(m_prev, ids_p, dloc_p, hbuf_p)

        return m

    def pair(t, m_prev):
        m_a = half_step(2 * t, m_prev, dbuf0, dbuf1, ids0, dloc0, hbuf0,
                        ids1, dloc1, hbuf1)
        m_b = half_step(2 * t + 1, m_a, dbuf1, dbuf0, ids1, dloc1, hbuf1,
                        ids0, dloc0, hbuf0)
        return m_b

    m_last = lax.fori_loop(0, NCH_M // 2, pair, 0)

    @pl.when(m_last > 0)
    def _():
        pltpu.make_async_copy(
            h_hbm.at[ids1.at[pl.ds(0, G)]], hbuf1, sem_h).wait()
        update_chunk(m_last, ids1, dloc1, hbuf1)

    pltpu.sync_copy(table, agg_hbm.at[pl.ds(lo, RPT)])


def _sc_smax(H, dst):
    return pl.kernel(
        _sc_smax_body,
        out_type=jax.ShapeDtypeStruct((NPAD, 64), jnp.float32),
        mesh=_mesh(),
        scratch_types=[
            pltpu.VMEM((RPT, 64), jnp.float32),
            pltpu.VMEM((CH,), jnp.int32),
            pltpu.VMEM((CH,), jnp.int32),
            pltpu.VMEM((CH,), jnp.int32),
            pltpu.VMEM((CH,), jnp.int32),
            pltpu.VMEM((CH,), jnp.int32),
            pltpu.VMEM((CH,), jnp.int32),
            pltpu.VMEM((G, 64), jnp.float32),
            pltpu.VMEM((G, 64), jnp.float32),
            pltpu.SemaphoreType.DMA,
            pltpu.SemaphoreType.DMA,
            pltpu.SemaphoreType.DMA,
        ],
        **_SC_PARAMS,
    )(H, dst)


def _tc_out_body(a_ref, wo_ref, bo_ref, o_ref):
    o = jnp.dot(a_ref[...], wo_ref[...], preferred_element_type=jnp.float32)
    o_ref[...] = jnp.maximum(o + bo_ref[...], 0.0)


def _tc_out(agg, Wo, bo):
    blk = 1024
    return pl.pallas_call(
        _tc_out_body,
        grid=(NPAD // blk,),
        in_specs=[
            pl.BlockSpec((blk, 64), lambda i: (i, 0)),
            pl.BlockSpec((64, 64), lambda i: (0, 0)),
            pl.BlockSpec((1, 64), lambda i: (0, 0)),
        ],
        out_specs=pl.BlockSpec((blk, 64), lambda i: (i, 0)),
        out_shape=jax.ShapeDtypeStruct((NPAD, 64), jnp.float32),
    )(agg, Wo, bo.reshape(1, 64))


def kernel(features, coordinates, keypoints, set_indices, W1, b1, W2, b2, Wo, bo):
    f_pad = jnp.pad(features, ((0, NPAD - N), (0, 0)))
    c_pad = jnp.pad(coordinates, ((0, NPAD - N), (0, 0)))
    kp_pad = jnp.pad(keypoints[:, 0], (0, NPAD - K))
    src = set_indices[0]
    dst = set_indices[1]

    W2bd = jnp.zeros((128, 256), jnp.float32)
    for i in range(4):
        W2bd = W2bd.at[32 * i:32 * (i + 1), 64 * i:64 * (i + 1)].set(W2)
    b2t = jnp.concatenate([b2, b2, b2, b2])

    P, RN = _tc_pr(f_pad, c_pad, W1, b1)
    QN = _sc_q(RN, kp_pad)
    E = _sc_edge(P, QN, src, dst)
    E4 = E.reshape(S // 4, 128)
    H2 = _tc_h(E4, W2bd, b2t)
    H = H2.reshape(S, 64)
    agg = _sc_smax(H, dst)
    out = _tc_out(agg, Wo, bo)
    return out[:K]


# speedup vs baseline: 9.3234x; 9.3234x over previous
import functools

import jax
import jax.numpy as jnp
from jax import lax
from jax.experimental import pallas as pl
from jax.experimental.pallas import tpu as pltpu
from jax.experimental.pallas import tpu_sc as plsc

N = 50000
K = 50000
S = 800000

NC, NS, LANES = 2, 16, 16
NW = NC * NS
RPT = 1568
NPAD = NW * RPT

CE = 1280
NCH_E = S // CE
ECH_PER_TILE = -(-NCH_E // NW)
SUBB = 128

CH = 3200
NCH_M = S // CH
G = 128

_SC_PARAMS = dict(
    compiler_params=pltpu.CompilerParams(
        use_tc_tiling_on_sc=False, needs_layout_passes=False
    ),
)


def _mesh():
    return plsc.VectorSubcoreMesh(
        core_axis_name="c", subcore_axis_name="s", num_cores=NC, num_subcores=NS
    )


def _wid():
    return lax.axis_index("s") * NC + lax.axis_index("c")


def _tc_pr_body(f_ref, c_ref, w1_ref, b1_ref, p_ref, rn_ref):
    c = c_ref[...]
    w1 = w1_ref[...]
    r = (c[:, 0:1] * w1[1:2, :] + c[:, 1:2] * w1[2:3, :] + c[:, 2:3] * w1[3:4, :])
    rn_ref[...] = -r
    p_ref[...] = r + f_ref[...] * w1[0:1, :] + b1_ref[...]


def _tc_pr(f_pad, c_pad, W1, b1):
    blk = 1024
    return pl.pallas_call(
        _tc_pr_body,
        grid=(NPAD // blk,),
        in_specs=[
            pl.BlockSpec((blk, 1), lambda i: (i, 0)),
            pl.BlockSpec((blk, 3), lambda i: (i, 0)),
            pl.BlockSpec((4, 32), lambda i: (0, 0)),
            pl.BlockSpec((1, 32), lambda i: (0, 0)),
        ],
        out_specs=[pl.BlockSpec((blk, 32), lambda i: (i, 0))] * 2,
        out_shape=[jax.ShapeDtypeStruct((NPAD, 32), jnp.float32)] * 2,
    )(f_pad, c_pad, W1, b1.reshape(1, 32))


def _sc_q_body(rn_hbm, kp_hbm, qn_hbm, idx_v, rows_v, sem):
    base = _wid() * RPT
    pltpu.sync_copy(kp_hbm.at[pl.ds(base, RPT)], idx_v)
    descs = []
    for b in range(RPT // 112):
        descs.append(
            pltpu.async_copy(
                rn_hbm.at[idx_v.at[pl.ds(b * 112, 112)]],
                rows_v.at[pl.ds(b * 112, 112)],
                sem,
            )
        )
    for d in descs:
        d.wait()
    pltpu.sync_copy(rows_v, qn_hbm.at[pl.ds(base, RPT)])


def _sc_q(RN, kp_pad):
    return pl.kernel(
        _sc_q_body,
        out_type=jax.ShapeDtypeStruct((NPAD, 32), jnp.float32),
        mesh=_mesh(),
        scratch_types=[
            pltpu.VMEM((RPT,), jnp.int32),
            pltpu.VMEM((RPT, 32), jnp.float32),
            pltpu.SemaphoreType.DMA,
        ],
        **_SC_PARAMS,
    )(RN, kp_pad)


def _sc_edge_body(p_hbm, qn_hbm, src_hbm, dst_hbm, e_hbm,
                  sidx, didx, eb, sem_i, sem_p, sem_q):
    w = _wid()

    def chunk_body(k, carry):
        c = w + k * NW

        @pl.when(c < NCH_E)
        def _():
            ebase = c * CE
            di = pltpu.async_copy(src_hbm.at[pl.ds(ebase, CE)], sidx, sem_i)
            dj = pltpu.async_copy(dst_hbm.at[pl.ds(ebase, CE)], didx, sem_i)
            di.wait()
            dj.wait()
            descs = []
            for b in range(CE // SUBB):
                descs.append(pltpu.async_copy(
                    p_hbm.at[sidx.at[pl.ds(b * SUBB, SUBB)]],
                    eb.at[pl.ds(b * SUBB, SUBB)], sem_p))
            for d in descs:
                d.wait()
            descs = []
            for b in range(CE // SUBB):
                d = pltpu.make_async_copy(
                    qn_hbm.at[didx.at[pl.ds(b * SUBB, SUBB)]],
                    eb.at[pl.ds(b * SUBB, SUBB)], sem_q)
                d.start(add=True)
                descs.append(d)
            for d in descs:
                d.wait()
            pltpu.sync_copy(eb, e_hbm.at[pl.ds(ebase, CE)])

        return carry

    lax.fori_loop(0, ECH_PER_TILE, chunk_body, 0)


def _sc_edge(P, QN, src, dst):
    return pl.kernel(
        _sc_edge_body,
        out_type=jax.ShapeDtypeStruct((S, 32), jnp.float32),
        mesh=_mesh(),
        scratch_types=[
            pltpu.VMEM((CE,), jnp.int32),
            pltpu.VMEM((CE,), jnp.int32),
            pltpu.VMEM((CE, 32), jnp.float32),
            pltpu.SemaphoreType.DMA,
            pltpu.SemaphoreType.DMA,
            pltpu.SemaphoreType.DMA,
        ],
        **_SC_PARAMS,
    )(P, QN, src, dst)


def _tc_h_body(e_ref, w2_ref, b2_ref, h_ref):
    x = jnp.maximum(e_ref[...], 0.0)
    h = jnp.dot(x, w2_ref[...], preferred_element_type=jnp.float32)
    h = jnp.maximum(h + b2_ref[...], 0.0)
    h_ref[...] = h.reshape(h_ref.shape)


def _tc_h(E4, W2bd, b2t):
    blk = 2000
    return pl.pallas_call(
        _tc_h_body,
        grid=(S // 4 // blk,),
        in_specs=[
            pl.BlockSpec((blk, 128), lambda i: (i, 0)),
            pl.BlockSpec((128, 256), lambda i: (0, 0)),
            pl.BlockSpec((1, 256), lambda i: (0, 0)),
        ],
        out_specs=pl.BlockSpec((2 * blk, 128), lambda i: (i, 0)),
        out_shape=jax.ShapeDtypeStruct((S // 2, 128), jnp.float32),
    )(E4, W2bd, b2t.reshape(1, 256))


def _sc_smax_body(h_hbm, dst_hbm, agg_hbm, table, dbuf0, dbuf1, ids, dloc,
                  hbuf0, hbuf1, spstage, sdloc, sem_d, sem_h):
    sid = lax.axis_index("s")
    w = _wid()
    lo = w * RPT
    zf = jnp.zeros((16,), jnp.float32)
    zi = jnp.zeros((16,), jnp.int32)

    def zrow(r, carry):
        for j in range(4):
            table[r, pl.ds(j * 16, 16)] = zf
        return carry

    lax.fori_loop(0, RPT, zrow, 0)

    def zids(v, carry):
        ids[pl.ds(v * 16, 16)] = zi
        return carry

    lax.fori_loop(0, CH // 16, zids, 0)

    iota = lax.iota(jnp.int32, 16)
    urpt = jnp.uint32(RPT)

    pltpu.async_copy(dst_hbm.at[pl.ds(0, CH)], dbuf0, sem_d)

    def upd_loop(hbuf, cnt):
        def upd1(i, carry3):
            d = sdloc[i]
            for j in range(4):
                t = table[d, pl.ds(j * 16, 16)]
                hh = hbuf[i, pl.ds(j * 16, 16)]
                table[d, pl.ds(j * 16, 16)] = jnp.maximum(t, hh)
            return carry3

        def updq(q, carry3):
            for k in range(4):
                upd1(q * 4 + k, 0)
            return carry3

        nq = cnt >> 2
        lax.fori_loop(0, nq, updq, 0)
        lax.fori_loop(nq * 4, cnt, upd1, 0)

    def do_chunk(cidx, dbuf_cur, dbuf_nxt):
        ebase = cidx * CH
        pltpu.make_async_copy(dst_hbm.at[pl.ds(ebase, CH)], dbuf_cur, sem_d).wait()

        @pl.when(cidx + 1 < NCH_M)
        def _():
            pltpu.async_copy(
                dst_hbm.at[pl.ds((cidx + 1) * CH, CH)], dbuf_nxt, sem_d)

        base_eid = ebase + iota

        def filt(v, m_v):
            for u in range(2):
                vv = v * 2 + u
                d16 = dbuf_cur[pl.ds(vv * 16, 16)]
                dl = d16 - lo
                msk = dl.astype(jnp.uint32) < urpt
                eid = base_eid + vv * 16
                csum = plsc.cumsum(msk.astype(jnp.int32))
                pos = csum + (m_v - 1)
                plsc.store_scatter(ids, [pos], eid, mask=msk)
                plsc.store_scatter(dloc, [pos], dl, mask=msk)
                m_v = m_v + plsc.all_reduce_population_count(msk)
            return m_v

        m_v = lax.fori_loop(0, CH // 32, filt, jnp.zeros((16,), jnp.int32))
        m = jnp.max(m_v)
        nb = (m + (G - 1)) >> 7

        @pl.when(nb > 0)
        def _():
            pltpu.async_copy(h_hbm.at[ids.at[pl.ds(0, G)]], hbuf0, sem_h)

        def sub(b, carry2):
            off = b * G
            cnt = jnp.minimum(G, m - off)
            pltpu.sync_copy(dloc.at[pl.ds(off, G)], spstage.at[sid])
            pltpu.sync_copy(spstage.at[sid], sdloc)
            par = b & 1

            @pl.when(par == 0)
            def _():
                pltpu.make_async_copy(
                    h_hbm.at[ids.at[pl.ds(off, G)]], hbuf0, sem_h).wait()

                @pl.when(b + 1 < nb)
                def _():
                    pltpu.async_copy(
                        h_hbm.at[ids.at[pl.ds(off + G, G)]], hbuf1, sem_h)

                upd_loop(hbuf0, cnt)

            @pl.when(par == 1)
            def _():
                pltpu.make_async_copy(
                    h_hbm.at[ids.at[pl.ds(off, G)]], hbuf1, sem_h).wait()

                @pl.when(b + 1 < nb)
                def _():
                    pltpu.async_copy(
                        h_hbm.at[ids.at[pl.ds(off + G, G)]], hbuf0, sem_h)

                upd_loop(hbuf1, cnt)

            return carry2

        lax.fori_loop(0, nb, sub, 0)

    def chunk(cidx, carry):
        par = cidx & 1

        @pl.when(par == 0)
        def _():
            do_chunk(cidx, dbuf0, dbuf1)

        @pl.when(par == 1)
        def _():
            do_chunk(cidx, dbuf1, dbuf0)

        return carry

    lax.fori_loop(0, NCH_M, chunk, 0)
    pltpu.sync_copy(table, agg_hbm.at[pl.ds(lo, RPT)])


def _sc_smax(H, dst):
    return pl.kernel(
        _sc_smax_body,
        out_type=jax.ShapeDtypeStruct((NPAD, 64), jnp.float32),
        mesh=_mesh(),
        scratch_types=[
            pltpu.VMEM((RPT, 64), jnp.float32),
            pltpu.VMEM((CH,), jnp.int32),
            pltpu.VMEM((CH,), jnp.int32),
            pltpu.VMEM((CH,), jnp.int32),
            pltpu.VMEM((CH,), jnp.int32),
            pltpu.VMEM((G, 64), jnp.float32),
            pltpu.VMEM((G, 64), jnp.float32),
            pltpu.VMEM_SHARED((NS, G), jnp.int32),
            pltpu.SMEM((G,), jnp.int32),
            pltpu.SemaphoreType.DMA,
            pltpu.SemaphoreType.DMA,
        ],
        **_SC_PARAMS,
    )(H, dst)


def _tc_out_body(a_ref, wo_ref, bo_ref, o_ref):
    o = jnp.dot(a_ref[...], wo_ref[...], preferred_element_type=jnp.float32)
    o_ref[...] = jnp.maximum(o + bo_ref[...], 0.0)


def _tc_out(agg, Wo, bo):
    blk = 1024
    return pl.pallas_call(
        _tc_out_body,
        grid=(NPAD // blk,),
        in_specs=[
            pl.BlockSpec((blk, 64), lambda i: (i, 0)),
            pl.BlockSpec((64, 64), lambda i: (0, 0)),
            pl.BlockSpec((1, 64), lambda i: (0, 0)),
        ],
        out_specs=pl.BlockSpec((blk, 64), lambda i: (i, 0)),
        out_shape=jax.ShapeDtypeStruct((NPAD, 64), jnp.float32),
    )(agg, Wo, bo.reshape(1, 64))


def kernel(features, coordinates, keypoints, set_indices, W1, b1, W2, b2, Wo, bo):
    f_pad = jnp.pad(features, ((0, NPAD - N), (0, 0)))
    c_pad = jnp.pad(coordinates, ((0, NPAD - N), (0, 0)))
    kp_pad = jnp.pad(keypoints[:, 0], (0, NPAD - K))
    src = set_indices[0]
    dst = set_indices[1]

    W2bd = jnp.zeros((128, 256), jnp.float32)
    for i in range(4):
        W2bd = W2bd.at[32 * i:32 * (i + 1), 64 * i:64 * (i + 1)].set(W2)
    b2t = jnp.concatenate([b2, b2, b2, b2])

    P, RN = _tc_pr(f_pad, c_pad, W1, b1)
    QN = _sc_q(RN, kp_pad)
    E = _sc_edge(P, QN, src, dst)
    E4 = E.reshape(S // 4, 128)
    H2 = _tc_h(E4, W2bd, b2t)
    H = H2.reshape(S, 64)
    agg = _sc_smax(H, dst)
    out = _tc_out(agg, Wo, bo)
    return out[:K]
